# Initial kernel scaffold; baseline (speedup 1.0000x reference)
import jax
import jax.numpy as jnp
from jax.experimental import pallas as pl


def kernel(x, connections, memory_words):
    raise NotImplementedError("write your pallas kernel here")


# speedup vs baseline: 9.6891x; 9.6891x over previous
import dataclasses
import functools

import jax
import jax.numpy as jnp
from jax import lax
from jax.experimental import pallas as pl
from jax.experimental.pallas import tpu as pltpu
from jax.experimental.pallas import tpu_sc as plsc

TOTAL_INPUT_BITS = 2048
NUM_NEURONS = 2048
N_BITS = 16
CELLS_PER_WORD = 31
WORDS_PER_NEURON = 2115
ROW32 = 2 * WORDS_PER_NEURON
BATCH = 4096

WB_BLK = 256
M_BLK = 256
N_PER_TILE = 16
B_CHUNK = 512


def _wbuild_body(c_ref, whi_ref, wlo_ref):
    cols = lax.broadcasted_iota(jnp.int32, (WB_BLK, TOTAL_INPUT_BITS), 1)
    hi = jnp.zeros((WB_BLK, TOTAL_INPUT_BITS), jnp.float32)
    lo = jnp.zeros((WB_BLK, TOTAL_INPUT_BITS), jnp.float32)
    for k in range(N_BITS):
        ck = c_ref[:, k]
        eq = cols == ck[:, None]
        if k < 8:
            hi = hi + jnp.where(eq, jnp.float32(2.0 ** (7 - k)),
                                jnp.float32(0.0))
        else:
            lo = lo + jnp.where(eq, jnp.float32(2.0 ** (15 - k)),
                                jnp.float32(0.0))
    whi_ref[...] = hi.astype(jnp.bfloat16)
    wlo_ref[...] = lo.astype(jnp.bfloat16)


def _build_weights(conn):
    return pl.pallas_call(
        _wbuild_body,
        grid=(NUM_NEURONS // WB_BLK,),
        in_specs=[pl.BlockSpec((WB_BLK, N_BITS), lambda i: (i, i * 0))],
        out_specs=[
            pl.BlockSpec((WB_BLK, TOTAL_INPUT_BITS), lambda i: (i, i * 0)),
            pl.BlockSpec((WB_BLK, TOTAL_INPUT_BITS), lambda i: (i, i * 0)),
        ],
        out_shape=[
            jax.ShapeDtypeStruct((NUM_NEURONS, TOTAL_INPUT_BITS), jnp.bfloat16),
            jax.ShapeDtypeStruct((NUM_NEURONS, TOTAL_INPUT_BITS), jnp.bfloat16),
        ],
    )(conn)


def _addr_body(x_ref, whi_ref, wlo_ref, out_ref):
    bits = jnp.where(x_ref[...] > jnp.float32(0.5), jnp.float32(1.0),
                     jnp.float32(0.0)).astype(jnp.bfloat16)
    dn = (((1,), (1,)), ((), ()))
    hi = lax.dot_general(whi_ref[...], bits, dn,
                         preferred_element_type=jnp.float32)
    lo = lax.dot_general(wlo_ref[...], bits, dn,
                         preferred_element_type=jnp.float32)
    af = hi * jnp.float32(256.0) + lo
    wi = ((af + jnp.float32(0.5)) * jnp.float32(1.0 / 31.0)).astype(jnp.int32)
    out_ref[...] = af.astype(jnp.int32) + wi


def _addresses(x, whi, wlo):
    return pl.pallas_call(
        _addr_body,
        grid=(BATCH // M_BLK,),
        in_specs=[
            pl.BlockSpec((M_BLK, TOTAL_INPUT_BITS), lambda j: (j, j * 0)),
            pl.BlockSpec((NUM_NEURONS, TOTAL_INPUT_BITS), lambda j: (j * 0, j * 0)),
            pl.BlockSpec((NUM_NEURONS, TOTAL_INPUT_BITS), lambda j: (j * 0, j * 0)),
        ],
        out_specs=pl.BlockSpec((NUM_NEURONS, M_BLK), lambda j: (j * 0, j)),
        out_shape=jax.ShapeDtypeStruct((NUM_NEURONS, BATCH), jnp.int32),
    )(x, whi, wlo)


def _sc_compiler_params():
    cp = pltpu.CompilerParams()
    if "needs_layout_passes" in pltpu.CompilerParams.__dataclass_fields__:
        cp = dataclasses.replace(cp, needs_layout_passes=False)
    return cp


def _sc_gather(comb_t, tab):
    mesh = plsc.VectorSubcoreMesh(core_axis_name="c", subcore_axis_name="s")

    @functools.partial(
        pl.kernel,
        out_type=jax.ShapeDtypeStruct((NUM_NEURONS, BATCH), jnp.float32),
        mesh=mesh,
        compiler_params=_sc_compiler_params(),
        scratch_types=[
            pltpu.VMEM((N_PER_TILE, ROW32), jnp.int32),
            pltpu.VMEM((N_PER_TILE, B_CHUNK), jnp.int32),
            pltpu.VMEM((N_PER_TILE, B_CHUNK), jnp.float32),
        ],
    )
    def k(comb_hbm, tab_hbm, out_hbm, tbuf, cbuf, obuf):
        wid = (lax.axis_index("s").astype(jnp.int32) * jnp.int32(2)
               + lax.axis_index("c").astype(jnp.int32))
        lanes = lax.iota(jnp.int32, 16)
        zeros16 = jnp.zeros((16,), jnp.int32)
        i32 = jnp.int32

        def vec_body(j, s):
            b = j * i32(16)
            v = cbuf[s, pl.ds(b, 16)]
            eidx = jnp.right_shift(v, i32(4))
            word = plsc.load_gather(tbuf, [zeros16 + s, eidx])
            sh = jnp.left_shift(jnp.bitwise_and(v, i32(15)), i32(1))
            cell = jnp.bitwise_and(jnp.right_shift(word, sh), i32(3))
            obuf[s, pl.ds(b, 16)] = cell.astype(jnp.float32)
            return s

        def neuron_body(s, carry):
            lax.fori_loop(i32(0), i32(B_CHUNK // 16), vec_body, s)
            return carry

        def chunk_body(cbk, n0):
            b0 = cbk * i32(B_CHUNK)
            pltpu.sync_copy(
                comb_hbm.at[pl.ds(n0, N_PER_TILE), pl.ds(b0, B_CHUNK)], cbuf)
            lax.fori_loop(i32(0), i32(N_PER_TILE), neuron_body, i32(0))
            pltpu.sync_copy(
                obuf, out_hbm.at[pl.ds(n0, N_PER_TILE), pl.ds(b0, B_CHUNK)])
            return n0

        def pass_body(p, carry):
            n0 = (p * i32(32) + wid) * i32(N_PER_TILE)
            pltpu.sync_copy(tab_hbm.at[pl.ds(n0, N_PER_TILE), :], tbuf)
            lax.fori_loop(i32(0), i32(BATCH // B_CHUNK), chunk_body, n0)
            return carry

        lax.fori_loop(i32(0), i32(NUM_NEURONS // (32 * N_PER_TILE)),
                      pass_body, i32(0))

    return k(comb_t, tab)


def kernel(x, connections, memory_words):
    conn = connections.astype(jnp.int32)
    tab = lax.bitcast_convert_type(memory_words, jnp.int32)
    tab = tab.reshape(NUM_NEURONS, ROW32)
    whi, wlo = _build_weights(conn)
    comb_t = _addresses(x, whi, wlo)
    out_t = _sc_gather(comb_t, tab)
    return out_t.T


# speedup vs baseline: 10.5806x; 1.0920x over previous
import dataclasses
import functools

import jax
import jax.numpy as jnp
from jax import lax
from jax.experimental import pallas as pl
from jax.experimental.pallas import tpu as pltpu
from jax.experimental.pallas import tpu_sc as plsc

TOTAL_INPUT_BITS = 2048
NUM_NEURONS = 2048
N_BITS = 16
CELLS_PER_WORD = 31
WORDS_PER_NEURON = 2115
ROW32 = 2 * WORDS_PER_NEURON
BATCH = 4096

WB_BLK = 256
M_BLK = 256
N_PER_TILE = 16
B_CHUNK = 512


def _wbuild_body(c_ref, whi_ref, wlo_ref):
    cols = lax.broadcasted_iota(jnp.int32, (WB_BLK, TOTAL_INPUT_BITS), 1)
    hi = jnp.zeros((WB_BLK, TOTAL_INPUT_BITS), jnp.float32)
    lo = jnp.zeros((WB_BLK, TOTAL_INPUT_BITS), jnp.float32)
    for k in range(N_BITS):
        ck = c_ref[:, k]
        eq = cols == ck[:, None]
        if k < 8:
            hi = hi + jnp.where(eq, jnp.float32(2.0 ** (7 - k)),
                                jnp.float32(0.0))
        else:
            lo = lo + jnp.where(eq, jnp.float32(2.0 ** (15 - k)),
                                jnp.float32(0.0))
    whi_ref[...] = hi.astype(jnp.bfloat16)
    wlo_ref[...] = lo.astype(jnp.bfloat16)


def _build_weights(conn):
    return pl.pallas_call(
        _wbuild_body,
        grid=(NUM_NEURONS // WB_BLK,),
        in_specs=[pl.BlockSpec((WB_BLK, N_BITS), lambda i: (i, i * 0))],
        out_specs=[
            pl.BlockSpec((WB_BLK, TOTAL_INPUT_BITS), lambda i: (i, i * 0)),
            pl.BlockSpec((WB_BLK, TOTAL_INPUT_BITS), lambda i: (i, i * 0)),
        ],
        out_shape=[
            jax.ShapeDtypeStruct((NUM_NEURONS, TOTAL_INPUT_BITS), jnp.bfloat16),
            jax.ShapeDtypeStruct((NUM_NEURONS, TOTAL_INPUT_BITS), jnp.bfloat16),
        ],
    )(conn)


def _addr_body(x_ref, whi_ref, wlo_ref, out_ref):
    bits = jnp.where(x_ref[...] > jnp.float32(0.5), jnp.float32(1.0),
                     jnp.float32(0.0)).astype(jnp.bfloat16)
    dn = (((1,), (1,)), ((), ()))
    hi = lax.dot_general(whi_ref[...], bits, dn,
                         preferred_element_type=jnp.float32)
    lo = lax.dot_general(wlo_ref[...], bits, dn,
                         preferred_element_type=jnp.float32)
    af = hi * jnp.float32(256.0) + lo
    wi = ((af + jnp.float32(0.5)) * jnp.float32(1.0 / 31.0)).astype(jnp.int32)
    out_ref[...] = af.astype(jnp.int32) + wi


def _addresses(x, whi, wlo):
    return pl.pallas_call(
        _addr_body,
        grid=(BATCH // M_BLK,),
        in_specs=[
            pl.BlockSpec((M_BLK, TOTAL_INPUT_BITS), lambda j: (j, j * 0)),
            pl.BlockSpec((NUM_NEURONS, TOTAL_INPUT_BITS), lambda j: (j * 0, j * 0)),
            pl.BlockSpec((NUM_NEURONS, TOTAL_INPUT_BITS), lambda j: (j * 0, j * 0)),
        ],
        out_specs=pl.BlockSpec((NUM_NEURONS, M_BLK), lambda j: (j * 0, j)),
        out_shape=jax.ShapeDtypeStruct((NUM_NEURONS, BATCH), jnp.int32),
    )(x, whi, wlo)


def _sc_compiler_params():
    cp = pltpu.CompilerParams()
    if "needs_layout_passes" in pltpu.CompilerParams.__dataclass_fields__:
        cp = dataclasses.replace(cp, needs_layout_passes=False)
    return cp


def _sc_gather(comb_t, tab):
    mesh = plsc.VectorSubcoreMesh(core_axis_name="c", subcore_axis_name="s")

    @functools.partial(
        pl.kernel,
        out_type=jax.ShapeDtypeStruct((NUM_NEURONS, BATCH), jnp.float32),
        mesh=mesh,
        compiler_params=_sc_compiler_params(),
        scratch_types=[
            pltpu.VMEM((N_PER_TILE, ROW32), jnp.int32),
            pltpu.VMEM((N_PER_TILE, B_CHUNK), jnp.int32),
            pltpu.VMEM((N_PER_TILE, B_CHUNK), jnp.float32),
        ],
    )
    def k(comb_hbm, tab_hbm, out_hbm, tbuf, cbuf, obuf):
        wid = (lax.axis_index("s").astype(jnp.int32) * jnp.int32(2)
               + lax.axis_index("c").astype(jnp.int32))
        lanes = lax.iota(jnp.int32, 16)
        zeros16 = jnp.zeros((16,), jnp.int32)
        i32 = jnp.int32

        def neuron_body(s, carry):
            sidx = zeros16 + s

            @plsc.parallel_loop(i32(0), i32(B_CHUNK // 16), i32(1), unroll=8)
            def _(j):
                b = j.astype(jnp.int32) * i32(16)
                v = cbuf[s, pl.ds(b, 16)]
                eidx = jnp.right_shift(v, i32(4))
                word = plsc.load_gather(tbuf, [sidx, eidx])
                sh = jnp.left_shift(jnp.bitwise_and(v, i32(15)), i32(1))
                cell = jnp.bitwise_and(jnp.right_shift(word, sh), i32(3))
                obuf[s, pl.ds(b, 16)] = cell.astype(jnp.float32)

            return carry

        def chunk_body(cbk, n0):
            b0 = cbk * i32(B_CHUNK)
            pltpu.sync_copy(
                comb_hbm.at[pl.ds(n0, N_PER_TILE), pl.ds(b0, B_CHUNK)], cbuf)
            lax.fori_loop(i32(0), i32(N_PER_TILE), neuron_body, i32(0))
            pltpu.sync_copy(
                obuf, out_hbm.at[pl.ds(n0, N_PER_TILE), pl.ds(b0, B_CHUNK)])
            return n0

        def pass_body(p, carry):
            n0 = (p * i32(32) + wid) * i32(N_PER_TILE)
            pltpu.sync_copy(tab_hbm.at[pl.ds(n0, N_PER_TILE), :], tbuf)
            lax.fori_loop(i32(0), i32(BATCH // B_CHUNK), chunk_body, n0)
            return carry

        lax.fori_loop(i32(0), i32(NUM_NEURONS // (32 * N_PER_TILE)),
                      pass_body, i32(0))

    return k(comb_t, tab)


def kernel(x, connections, memory_words):
    conn = connections.astype(jnp.int32)
    tab = lax.bitcast_convert_type(memory_words, jnp.int32)
    tab = tab.reshape(NUM_NEURONS, ROW32)
    whi, wlo = _build_weights(conn)
    comb_t = _addresses(x, whi, wlo)
    out_t = _sc_gather(comb_t, tab)
    return out_t.T
